# baseline (speedup 1.0000x reference)
import functools

import numpy as np
import jax
import jax.numpy as jnp
from jax import lax
from jax.experimental import pallas as pl
from jax.experimental.pallas import tpu as pltpu
from jax.experimental.pallas import tpu_sc as plsc

_B, _S, _D = 4096, 200, 64
_V = 1000000
_NC, _NS = 2, 16
_NW = _NC * _NS
_SCALE = 8.0
_LANES = 16

_BLKS = _V // 128
_TAIL = _V - _BLKS * 128
_APW = _BLKS // _NW
_AREM = _BLKS - _APW * _NW

_BC = _B // 128
_ITEMS = _S * _BC
_IPW = _ITEMS // _NW
_NBUF = 4


def _make_pe() -> np.ndarray:
    pos = np.arange(_S, dtype=np.float32)[:, None]
    div = np.power(np.float32(10000.0),
                   np.arange(_D // 2, dtype=np.float32) * np.float32(2.0 / _D))
    pe = np.zeros((_S, _D), dtype=np.float32)
    pe[:, 0::2] = np.sin(pos / div)
    pe[:, 1::2] = np.cos(pos / div)
    return pe


_PE = _make_pe()

_mesh = plsc.VectorSubcoreMesh(core_axis_name="c", subcore_axis_name="s")


@functools.partial(
    pl.kernel,
    out_type=jax.ShapeDtypeStruct((_V // 2, 128), jnp.float32),
    mesh=_mesh,
    scratch_types=[
        pltpu.VMEM((2, _D, 128), jnp.float32),
        pltpu.VMEM((2, _D, 128), jnp.float32),
        pltpu.SemaphoreType.DMA,
        pltpu.SemaphoreType.DMA,
        pltpu.SemaphoreType.DMA,
        pltpu.SemaphoreType.DMA,
    ],
    compiler_params=pltpu.CompilerParams(use_tc_tiling_on_sc=True,
                                         needs_layout_passes=False),
)
def _detile(tt_hbm, tail_hbm, out_hbm, ibuf, obuf, rsem0, rsem1, wsem0,
            wsem1):
    rsem = (rsem0, rsem1)
    wsem = (wsem0, wsem1)
    wid = lax.axis_index("s") * _NC + lax.axis_index("c")
    base = wid * _APW + jnp.minimum(wid, _AREM)
    count = _APW + jnp.where(wid < _AREM, 1, 0)
    iota = jax.lax.iota(jnp.int32, _LANES)
    d_idx = [iota + 16 * kk for kk in range(4)]

    def start_read(i, b):
        col = (base + i) * 128
        pltpu.async_copy(tt_hbm.at[:, pl.ds(col, 128)], ibuf.at[b], rsem[b])

    def wait_read(b):
        pltpu.make_async_copy(tt_hbm.at[:, pl.ds(0, 128)], ibuf.at[b],
                              rsem[b]).wait()

    def start_write(i, b):
        row = (base + i) * 64
        pltpu.async_copy(obuf.at[b], out_hbm.at[pl.ds(row, 64)], wsem[b])

    def wait_write(b):
        pltpu.make_async_copy(obuf.at[b], out_hbm.at[pl.ds(0, 64)],
                              wsem[b]).wait()

    def transpose(b):
        @plsc.parallel_loop(0, 64, unroll=2)
        def _q(q):
            for k in range(8):
                j_idx = jnp.full((_LANES,), 2 * q + (1 if k >= 4 else 0),
                                 jnp.int32)
                val = plsc.load_gather(ibuf.at[b], [d_idx[k % 4], j_idx])
                obuf[b, q, pl.ds(16 * k, 16)] = val

    start_read(0, 0)
    start_read(1, 1)

    @pl.loop(0, _APW + 2, step=2)
    def _unit(ii):
        for b in range(2):
            i = ii + b

            @pl.when(i < count)
            def _():
                wait_read(b)

                @pl.when(i >= 2)
                def _():
                    wait_write(b)

                transpose(b)
                start_write(i, b)

                @pl.when(i + 2 < count)
                def _():
                    start_read(i + 2, b)

    wait_write(0)
    wait_write(1)

    @pl.when(wid == _NW - 1)
    def _tail():
        pltpu.sync_copy(tail_hbm, ibuf.at[0, pl.ds(0, 32)])
        pltpu.sync_copy(ibuf.at[0, pl.ds(0, 32)],
                        out_hbm.at[pl.ds(_BLKS * 64, _TAIL // 2)])


@functools.partial(
    pl.kernel,
    out_type=jax.ShapeDtypeStruct((_S, 8, _BC, 8, 128), jnp.float32),
    mesh=_mesh,
    scratch_types=[
        pltpu.VMEM((_NBUF, 128), jnp.int32),
        pltpu.VMEM((_NBUF, 128, _D), jnp.float32),
        pltpu.VMEM((_NBUF, 8, 8, 128), jnp.float32),
        pltpu.VMEM((_S, _D), jnp.float32),
        [pltpu.SemaphoreType.DMA] * _NBUF,
        [pltpu.SemaphoreType.DMA] * _NBUF,
        [pltpu.SemaphoreType.DMA] * _NBUF,
    ],
    compiler_params=pltpu.CompilerParams(use_tc_tiling_on_sc=False,
                                         needs_layout_passes=False),
)
def _embed(xt_hbm, pe_hbm, tl_hbm, out_hbm, idx_v, gbuf, obuf, pe_v,
           isem, gsem, osem):
    wid = lax.axis_index("s") * _NC + lax.axis_index("c")
    pltpu.sync_copy(pe_hbm, pe_v)
    mbase = wid * _IPW
    iota = jax.lax.iota(jnp.int32, _LANES)
    tok_idx = [iota + 16 * kk for kk in range(8)]

    def item_sb(i):
        m = mbase + i
        return m // _BC, m % _BC

    def start_idx(i, b):
        s, bc = item_sb(i)
        pltpu.async_copy(xt_hbm.at[s, pl.ds(bc * 128, 128)], idx_v.at[b],
                         isem[b])

    def wait_idx(b):
        pltpu.make_async_copy(xt_hbm.at[0, pl.ds(0, 128)], idx_v.at[b],
                              isem[b]).wait()

    def start_gather(b):
        for h in range(2):
            pltpu.async_copy(tl_hbm.at[idx_v.at[b, pl.ds(64 * h, 64)]],
                             gbuf.at[b, pl.ds(64 * h, 64)], gsem[b])

    def wait_gather(b):
        for h in range(2):
            pltpu.make_async_copy(tl_hbm.at[idx_v.at[b, pl.ds(64 * h, 64)]],
                                  gbuf.at[b, pl.ds(64 * h, 64)],
                                  gsem[b]).wait()

    def start_out(i, b):
        s, bc = item_sb(i)
        pltpu.async_copy(obuf.at[b], out_hbm.at[s, :, bc], osem[b])

    def wait_out(b):
        pltpu.make_async_copy(obuf.at[b], out_hbm.at[0, :, 0], osem[b]).wait()

    def compute(i, b):
        s, _ = item_sb(i)
        s_idx = jnp.full((_LANES,), s, jnp.int32)

        @plsc.parallel_loop(0, _D, unroll=2)
        def _dd(dd):
            dd_idx = jnp.full((_LANES,), dd, jnp.int32)
            pe_val = plsc.load_gather(pe_v, [s_idx, dd_idx])
            dr = dd // 8
            di = dd % 8
            for k in range(8):
                val = plsc.load_gather(gbuf.at[b], [tok_idx[k], dd_idx])
                obuf[b, dr, di, pl.ds(16 * k, 16)] = val * _SCALE + pe_val

    start_idx(0, 0)
    start_idx(1, 1)
    wait_idx(0)
    start_gather(0)
    wait_idx(1)
    start_gather(1)
    start_idx(2, 2)
    start_idx(3, 3)

    @pl.loop(0, _IPW, step=_NBUF)
    def _item(ii):
        for b in range(_NBUF):
            i = ii + b
            nb2 = (b + 2) % _NBUF

            @pl.when(i + 2 < _IPW)
            def _():
                wait_idx(nb2)
                start_gather(nb2)

            wait_gather(b)

            @pl.when(i + 4 < _IPW)
            def _():
                start_idx(i + 4, b)

            @pl.when(i >= _NBUF)
            def _():
                wait_out(b)

            compute(i, b)
            start_out(i, b)

    for b in range(_NBUF):
        wait_out(b)


def kernel(x, table):
    tt = table.T
    tail = table[_BLKS * 128:].reshape(_TAIL // 2, 128)
    t2 = _detile(tt, tail)
    tl = t2.reshape(_V, _D)
    xt = x.T
    pe = jnp.asarray(_PE)
    out5 = _embed(xt, pe, tl)
    out = out5.transpose(2, 4, 0, 1, 3).reshape(_B, _S, _D)
    return out


# speedup vs baseline: 2.0246x; 2.0246x over previous
import functools

import numpy as np
import jax
import jax.numpy as jnp
from jax import lax
from jax.experimental import pallas as pl
from jax.experimental.pallas import tpu as pltpu
from jax.experimental.pallas import tpu_sc as plsc

_B, _S, _D = 4096, 200, 64
_V = 1000000
_NC, _NS = 2, 16
_NW = _NC * _NS
_SCALE = 8.0
_LANES = 16

_BLKS = _V // 128
_TAIL = _V - _BLKS * 128
_APW = _BLKS // _NW
_AREM = _BLKS - _APW * _NW

_BC = _B // 128
_ITEMS = _S * _BC
_IPW = _ITEMS // _NW
_NBUF = 4


def _make_pe() -> np.ndarray:
    pos = np.arange(_S, dtype=np.float32)[:, None]
    div = np.power(np.float32(10000.0),
                   np.arange(_D // 2, dtype=np.float32) * np.float32(2.0 / _D))
    pe = np.zeros((_S, _D), dtype=np.float32)
    pe[:, 0::2] = np.sin(pos / div)
    pe[:, 1::2] = np.cos(pos / div)
    return pe


_PE = _make_pe()

_mesh = plsc.VectorSubcoreMesh(core_axis_name="c", subcore_axis_name="s")

_AW = 2048
_AGRID = -(-_V // _AW)


def _detile_body(in_ref, out_ref):
    t3 = in_ref[...].T.reshape(_AW // 2, 2, _D)
    out_ref[...] = jnp.concatenate([t3[:, 0, :], t3[:, 1, :]], axis=1)


_detile = pl.pallas_call(
    _detile_body,
    out_shape=jax.ShapeDtypeStruct((_V // 2, 128), jnp.float32),
    grid=(_AGRID,),
    in_specs=[pl.BlockSpec((_D, _AW), lambda g: (0, g))],
    out_specs=pl.BlockSpec((_AW // 2, 128), lambda g: (g, 0)),
)


@functools.partial(
    pl.kernel,
    out_type=jax.ShapeDtypeStruct((_S, 8, _BC, 8, 128), jnp.float32),
    mesh=_mesh,
    scratch_types=[
        pltpu.VMEM((_NBUF, 128), jnp.int32),
        pltpu.VMEM((_NBUF, 128, _D), jnp.float32),
        pltpu.VMEM((_NBUF, 8, 8, 128), jnp.float32),
        pltpu.VMEM((128, _D + 1), jnp.float32),
        pltpu.VMEM((_S, _D), jnp.float32),
        [pltpu.SemaphoreType.DMA] * _NBUF,
        [pltpu.SemaphoreType.DMA] * _NBUF,
        [pltpu.SemaphoreType.DMA] * _NBUF,
    ],
    compiler_params=pltpu.CompilerParams(use_tc_tiling_on_sc=False,
                                         needs_layout_passes=False),
)
def _embed(xt_hbm, pe_hbm, tl_hbm, out_hbm, idx_v, gbuf, obuf, tbuf, pe_v,
           isem, gsem, osem):
    wid = lax.axis_index("s") * _NC + lax.axis_index("c")
    pltpu.sync_copy(pe_hbm, pe_v)
    mbase = wid * _IPW
    iota = jax.lax.iota(jnp.int32, _LANES)
    tok_idx = [iota + 16 * kk for kk in range(8)]

    def item_sb(i):
        m = mbase + i
        return m // _BC, m % _BC

    def start_idx(i, b):
        s, bc = item_sb(i)
        pltpu.async_copy(xt_hbm.at[s, pl.ds(bc * 128, 128)], idx_v.at[b],
                         isem[b])

    def wait_idx(b):
        pltpu.make_async_copy(xt_hbm.at[0, pl.ds(0, 128)], idx_v.at[b],
                              isem[b]).wait()

    def start_gather(b):
        for h in range(2):
            pltpu.async_copy(tl_hbm.at[idx_v.at[b, pl.ds(64 * h, 64)]],
                             gbuf.at[b, pl.ds(64 * h, 64)], gsem[b])

    def wait_gather(b):
        for h in range(2):
            pltpu.make_async_copy(tl_hbm.at[idx_v.at[b, pl.ds(64 * h, 64)]],
                                  gbuf.at[b, pl.ds(64 * h, 64)],
                                  gsem[b]).wait()

    def start_out(i, b):
        s, bc = item_sb(i)
        pltpu.async_copy(obuf.at[b], out_hbm.at[s, :, bc], osem[b])

    def wait_out(b):
        pltpu.make_async_copy(obuf.at[b], out_hbm.at[0, :, 0], osem[b]).wait()

    def compute(i, b):
        s, _ = item_sb(i)
        s_idx = jnp.full((_LANES,), s, jnp.int32)

        @plsc.parallel_loop(0, 128, unroll=2)
        def _tok(tok):
            for k in range(_D // _LANES):
                tbuf[tok, pl.ds(16 * k, 16)] = gbuf[b, tok, pl.ds(16 * k, 16)]

        @plsc.parallel_loop(0, _D, unroll=2)
        def _dd(dd):
            dd_idx = jnp.full((_LANES,), dd, jnp.int32)
            pe_val = plsc.load_gather(pe_v, [s_idx, dd_idx])
            dr = dd // 8
            di = dd % 8
            for k in range(8):
                val = plsc.load_gather(tbuf, [tok_idx[k], dd_idx])
                obuf[b, dr, di, pl.ds(16 * k, 16)] = val * _SCALE + pe_val

    start_idx(0, 0)
    start_idx(1, 1)
    wait_idx(0)
    start_gather(0)
    wait_idx(1)
    start_gather(1)
    start_idx(2, 2)
    start_idx(3, 3)

    @pl.loop(0, _IPW, step=_NBUF)
    def _item(ii):
        for b in range(_NBUF):
            i = ii + b
            nb2 = (b + 2) % _NBUF

            @pl.when(i + 2 < _IPW)
            def _():
                wait_idx(nb2)
                start_gather(nb2)

            wait_gather(b)

            @pl.when(i + 4 < _IPW)
            def _():
                start_idx(i + 4, b)

            @pl.when(i >= _NBUF)
            def _():
                wait_out(b)

            compute(i, b)
            start_out(i, b)

    for b in range(_NBUF):
        wait_out(b)


def kernel(x, table):
    tt = table.T
    t2 = _detile(tt)
    tl = t2.reshape(_V, _D)
    xt = x.T
    pe = jnp.asarray(_PE)
    out5 = _embed(xt, pe, tl)
    out = out5.transpose(2, 4, 0, 1, 3).reshape(_B, _S, _D)
    return out
